# Initial kernel scaffold; baseline (speedup 1.0000x reference)
import jax
import jax.numpy as jnp
from jax.experimental import pallas as pl


def kernel(x_source, x_target, W1_st_n, W1_st_r, b1_st, W1_ts_n, W1_ts_r, b1_ts, W2_st_n, W2_st_r, b2_st, W2_ts_n, W2_ts_r, b2_ts, W_lin, b_lin, edge_index_st, edge_index_ts):
    raise NotImplementedError("write your pallas kernel here")


# speedup vs baseline: 7.3305x; 7.3305x over previous
import jax
import jax.numpy as jnp
from jax import lax
from jax.experimental import pallas as pl
from jax.experimental.pallas import tpu as pltpu
from jax.experimental.pallas import tpu_sc as plsc

N = 10000
NP = 10240
D = 128
HD = D // 2
OUT = 64
E = 320000
EP = 327680
CH = 128
NC = 2
NS = 16
ROWS_PER_TILE = NP // NS
N_CHUNKS = EP // NS // CH
CNT_W = 16
F32 = jnp.float32


def _fill_rows(ref, nrows, ncols, val):
    def body(i, _):
        for l in range(ncols // 16):
            ref[i, pl.ds(l * 16, 16)] = jnp.full((16,), val, F32)
        return 0
    lax.fori_loop(0, nrows, body, 0)


def _seg_pipeline(table, src_v, rows_a, rows_b, sem_a, sem_b, scatter):
    pltpu.async_copy(table.at[src_v.at[0]], rows_a, sem_a)

    def pair(p, _):
        j0 = 2 * p
        pltpu.make_async_copy(table.at[pl.ds(0, CH)], rows_a, sem_a).wait()
        pltpu.async_copy(table.at[src_v.at[j0 + 1]], rows_b, sem_b)
        scatter(rows_a, j0)
        pltpu.make_async_copy(table.at[pl.ds(0, CH)], rows_b, sem_b).wait()

        @pl.when(j0 + 2 < N_CHUNKS)
        def _():
            pltpu.async_copy(table.at[src_v.at[j0 + 2]], rows_a, sem_a)

        scatter(rows_b, j0 + 1)
        return 0

    lax.fori_loop(0, N_CHUNKS // 2, pair, 0)


def _seg_cnt_body(table, src_idx, dst_idx, acc_out, cnt_out,
                  src_v, dst_v, rows_a, rows_b, ones_v, zb_v,
                  acc_sh, cnt_sh, sem_a, sem_b):
    c = lax.axis_index("c")
    s = lax.axis_index("s")

    _fill_rows(rows_a, CH, HD, 0.0)
    _fill_rows(ones_v, CH, CNT_W, 1.0)
    _fill_rows(zb_v, CH, CNT_W, 0.0)

    base = s * ROWS_PER_TILE
    for k in range(ROWS_PER_TILE // CH):
        pltpu.sync_copy(rows_a, acc_sh.at[pl.ds(base + k * CH, CH)])
        pltpu.sync_copy(zb_v, cnt_sh.at[pl.ds(base + k * CH, CH)])
    plsc.subcore_barrier()

    pltpu.sync_copy(src_idx.at[c, s], src_v)
    pltpu.sync_copy(dst_idx.at[s], dst_v)

    def scatter(rows, j):
        pltpu.sync_copy(rows, acc_sh.at[dst_v.at[j]], add=True)

        @pl.when(c == 0)
        def _():
            pltpu.sync_copy(ones_v, cnt_sh.at[dst_v.at[j]], add=True)

    _seg_pipeline(table, src_v, rows_a, rows_b, sem_a, sem_b, scatter)
    plsc.subcore_barrier()

    pltpu.sync_copy(acc_sh.at[pl.ds(base, ROWS_PER_TILE)],
                    acc_out.at[c, pl.ds(base, ROWS_PER_TILE)])

    @pl.when(c == 0)
    def _():
        pltpu.sync_copy(cnt_sh.at[pl.ds(base, ROWS_PER_TILE)],
                        cnt_out.at[pl.ds(base, ROWS_PER_TILE)])


def _seg_body(table, src_idx, dst_idx, acc_out,
              src_v, dst_v, rows_a, rows_b, acc_sh, sem_a, sem_b):
    c = lax.axis_index("c")
    s = lax.axis_index("s")

    _fill_rows(rows_a, CH, HD, 0.0)
    base = s * ROWS_PER_TILE
    for k in range(ROWS_PER_TILE // CH):
        pltpu.sync_copy(rows_a, acc_sh.at[pl.ds(base + k * CH, CH)])
    plsc.subcore_barrier()

    pltpu.sync_copy(src_idx.at[c, s], src_v)
    pltpu.sync_copy(dst_idx.at[s], dst_v)

    def scatter(rows, j):
        pltpu.sync_copy(rows, acc_sh.at[dst_v.at[j]], add=True)

    _seg_pipeline(table, src_v, rows_a, rows_b, sem_a, sem_b, scatter)
    plsc.subcore_barrier()

    pltpu.sync_copy(acc_sh.at[pl.ds(base, ROWS_PER_TILE)],
                    acc_out.at[c, pl.ds(base, ROWS_PER_TILE)])


def _leaky(x):
    return jnp.where(x >= 0, x, 0.01 * x)


def _sage_block(acc_lo, acc_hi, cnt, x, wn, wr, b):
    mean = (jnp.concatenate([acc_lo, acc_hi], axis=1)
            / jnp.maximum(cnt[:, :1], 1.0))
    return (jnp.dot(mean, wn, preferred_element_type=F32)
            + jnp.dot(x, wr, preferred_element_type=F32) + b)


def _tc1_body(accst_lo, accst_hi, cnt_st, xt, w1stn, w1str, b1st,
              accts_lo, accts_hi, cnt_ts, xs, w1tsn, w1tsr, b1ts,
              xt_out, xs_out):
    xt_out[...] = _leaky(_sage_block(accst_lo[...], accst_hi[...],
                                     cnt_st[...], xt[...],
                                     w1stn[...], w1str[...], b1st[...]))
    xs_out[...] = _leaky(_sage_block(accts_lo[...], accts_hi[...],
                                     cnt_ts[...], xs[...],
                                     w1tsn[...], w1tsr[...], b1ts[...]))


def _tc2_body(acc_lo, acc_hi, cnt_ts, xs, w2tsn, w2tsr, b2ts, wlin, blin,
              out):
    h = _leaky(_sage_block(acc_lo[...], acc_hi[...], cnt_ts[...], xs[...],
                           w2tsn[...], w2tsr[...], b2ts[...]))
    out[...] = jnp.dot(h, wlin[...], preferred_element_type=F32) + blin[...]


_MESH = plsc.VectorSubcoreMesh(core_axis_name="c", subcore_axis_name="s")

_seg_cnt = pl.kernel(
    _seg_cnt_body,
    out_type=[jax.ShapeDtypeStruct((NC, NP, HD), F32),
              jax.ShapeDtypeStruct((NP, CNT_W), F32)],
    mesh=_MESH,
    scratch_types=[
        pltpu.VMEM((N_CHUNKS, CH), jnp.int32),
        pltpu.VMEM((N_CHUNKS, CH), jnp.int32),
        pltpu.VMEM((CH, HD), F32),
        pltpu.VMEM((CH, HD), F32),
        pltpu.VMEM((CH, CNT_W), F32),
        pltpu.VMEM((CH, CNT_W), F32),
        pltpu.VMEM_SHARED((NP, HD), F32),
        pltpu.VMEM_SHARED((NP, CNT_W), F32),
        pltpu.SemaphoreType.DMA,
        pltpu.SemaphoreType.DMA,
    ],
    compiler_params=pltpu.CompilerParams(use_tc_tiling_on_sc=False),
    name="sage_seg_cnt",
)

_seg = pl.kernel(
    _seg_body,
    out_type=[jax.ShapeDtypeStruct((NC, NP, HD), F32)],
    mesh=_MESH,
    scratch_types=[
        pltpu.VMEM((N_CHUNKS, CH), jnp.int32),
        pltpu.VMEM((N_CHUNKS, CH), jnp.int32),
        pltpu.VMEM((CH, HD), F32),
        pltpu.VMEM((CH, HD), F32),
        pltpu.VMEM_SHARED((NP, HD), F32),
        pltpu.SemaphoreType.DMA,
        pltpu.SemaphoreType.DMA,
    ],
    compiler_params=pltpu.CompilerParams(use_tc_tiling_on_sc=False),
    name="sage_seg",
)

_R = 512
_GRID = NP // _R

def _rowspec(width):
    return pl.BlockSpec((_R, width), lambda i: (i, 0))

def _fullspec(shape):
    return pl.BlockSpec(shape, lambda i: tuple(0 for _ in shape))

_tc1 = pl.pallas_call(
    _tc1_body,
    grid=(_GRID,),
    in_specs=[
        _rowspec(HD), _rowspec(HD), _rowspec(CNT_W), _rowspec(D),
        _fullspec((D, D)), _fullspec((D, D)), _fullspec((1, D)),
        _rowspec(HD), _rowspec(HD), _rowspec(CNT_W), _rowspec(D),
        _fullspec((D, D)), _fullspec((D, D)), _fullspec((1, D)),
    ],
    out_specs=[_rowspec(D), _rowspec(D)],
    out_shape=[jax.ShapeDtypeStruct((NP, D), F32),
               jax.ShapeDtypeStruct((NP, D), F32)],
    name="sage_tc1",
)

_tc2 = pl.pallas_call(
    _tc2_body,
    grid=(_GRID,),
    in_specs=[
        _rowspec(HD), _rowspec(HD), _rowspec(CNT_W), _rowspec(D),
        _fullspec((D, D)), _fullspec((D, D)), _fullspec((1, D)),
        _fullspec((D, OUT)), _fullspec((1, OUT)),
    ],
    out_specs=[_rowspec(OUT)],
    out_shape=[jax.ShapeDtypeStruct((NP, OUT), F32)],
    name="sage_tc2",
)


def _split_table(x):
    return jnp.concatenate([x[:, :HD], x[:, HD:]], axis=0)


def kernel(x_source, x_target,
           W1_st_n, W1_st_r, b1_st, W1_ts_n, W1_ts_r, b1_ts,
           W2_st_n, W2_st_r, b2_st, W2_ts_n, W2_ts_r, b2_ts,
           W_lin, b_lin,
           edge_index_st, edge_index_ts):
    xs_p = jnp.pad(x_source.astype(F32), ((0, NP - N), (0, 0)))
    xt_p = jnp.pad(x_target.astype(F32), ((0, NP - N), (0, 0)))

    pad_ids = (jnp.arange(EP - E, dtype=jnp.int32) % (NP - N)) + N

    def pad_e(v):
        return jnp.concatenate([v.astype(jnp.int32), pad_ids])

    st_src, st_dst = pad_e(edge_index_st[0]), pad_e(edge_index_st[1])
    ts_src, ts_dst = pad_e(edge_index_ts[0]), pad_e(edge_index_ts[1])

    def pack_src(v):
        v = v.reshape(NS, N_CHUNKS, CH)
        return jnp.stack([v, v + NP])

    src_st = pack_src(st_src)
    dst_st = st_dst.reshape(NS, N_CHUNKS, CH)
    src_ts = pack_src(ts_src)
    dst_ts = ts_dst.reshape(NS, N_CHUNKS, CH)

    acc_st, cnt_st = _seg_cnt(_split_table(xs_p), src_st, dst_st)
    acc_ts, cnt_ts = _seg_cnt(_split_table(xt_p), src_ts, dst_ts)

    xt_p2, xs_p2 = _tc1(
        acc_st[0], acc_st[1], cnt_st, xt_p,
        W1_st_n, W1_st_r, b1_st.reshape(1, D),
        acc_ts[0], acc_ts[1], cnt_ts, xs_p,
        W1_ts_n, W1_ts_r, b1_ts.reshape(1, D))

    (acc2,) = _seg(_split_table(xt_p2), src_ts, dst_ts)

    (out,) = _tc2(acc2[0], acc2[1], cnt_ts, xs_p2,
                  W2_ts_n, W2_ts_r, b2_ts.reshape(1, D),
                  W_lin, b_lin.reshape(1, OUT))
    return out[:N]
